# baseline (speedup 1.0000x reference)
import functools

import jax
import jax.numpy as jnp
from jax import lax
from jax.experimental import pallas as pl
from jax.experimental.pallas import tpu as pltpu
from jax.experimental.pallas import tpu_sc as plsc

NC = 2
NS = 16
NW = NC * NS
L = 16

D = 64
DP = 128
B = 4096
S = 200
RPW = B // NW
NBUF = 3
C0 = 104
C1 = S - C0
NCH = 2 * RPW

_mesh = plsc.VectorSubcoreMesh(core_axis_name="c", subcore_axis_name="s")


@functools.partial(
    pl.kernel,
    mesh=_mesh,
    compiler_params=pltpu.CompilerParams(use_tc_tiling_on_sc=False),
    out_type=jax.ShapeDtypeStruct((B, S, DP), jnp.float32),
    scratch_types=[
        pltpu.VMEM((RPW, S), jnp.int32),
        pltpu.VMEM((RPW, S), jnp.float32),
        pltpu.VMEM((D,), jnp.float32),
        pltpu.VMEM((D,), jnp.float32),
        pltpu.VMEM((C0, D), jnp.float32),
        pltpu.VMEM((C0, D), jnp.float32),
        pltpu.VMEM((C0, D), jnp.float32),
        pltpu.VMEM((C0, DP), jnp.float32),
        pltpu.VMEM((C0, DP), jnp.float32),
        pltpu.VMEM((C0, DP), jnp.float32),
        pltpu.SemaphoreType.DMA,
        pltpu.SemaphoreType.DMA,
        pltpu.SemaphoreType.DMA,
        pltpu.SemaphoreType.DMA,
        pltpu.SemaphoreType.DMA,
        pltpu.SemaphoreType.DMA,
    ],
)
def _sc_tokenize(idx_hbm, expr_hbm, table_hbm, w_hbm, b_hbm, out_hbm,
                 idx_v, expr_v, w_v, b_v,
                 ga, gb, gc, oa, ob, oc,
                 g0, g1, g2, s0, s1, s2):
    gbuf = [ga, gb, gc]
    obuf = [oa, ob, oc]
    gsem = [g0, g1, g2]
    ssem = [s0, s1, s2]

    wid = lax.axis_index("s") * NC + lax.axis_index("c")
    row_base = wid * RPW

    pltpu.sync_copy(idx_hbm.at[pl.ds(row_base, RPW)], idx_v)
    pltpu.sync_copy(expr_hbm.at[pl.ds(row_base, RPW)], expr_v)
    pltpu.sync_copy(w_hbm, w_v)
    pltpu.sync_copy(b_hbm, b_v)

    wq = [w_v[pl.ds(q * L, L)] for q in range(4)]
    bq = [b_v[pl.ds(q * L, L)] for q in range(4)]

    def start_gather(j, s, par):
        row = j // 2
        off, n = (0, C0) if par == 0 else (C0, C1)
        pltpu.make_async_copy(
            table_hbm.at[idx_v.at[row, pl.ds(off, n)]],
            gbuf[s].at[pl.ds(0, n)], gsem[s]).start()

    def wait_gather(s, par):
        n = C0 if par == 0 else C1
        pltpu.make_async_copy(
            table_hbm.at[idx_v.at[0, pl.ds(0, n)]],
            gbuf[s].at[pl.ds(0, n)], gsem[s]).wait()

    def start_store(j, s, par):
        row = j // 2
        off, n = (0, C0) if par == 0 else (C0, C1)
        pltpu.make_async_copy(
            obuf[s].at[pl.ds(0, n)],
            out_hbm.at[row_base + row, pl.ds(off, n)], ssem[s]).start()

    def wait_store(s, par):
        n = C0 if par == 0 else C1
        pltpu.make_async_copy(
            obuf[s].at[pl.ds(0, n)],
            out_hbm.at[row_base, pl.ds(0, n)], ssem[s]).wait()

    def compute(j, s, par):
        row = j // 2
        eoff, n = (0, C0) if par == 0 else (C0, C1)
        g = gbuf[s]
        o = obuf[s]

        def do_row(ii, ev16, u):
            ev = jnp.full((L,), ev16[u], dtype=jnp.float32)
            for q in range(4):
                sl = pl.ds(q * L, L)
                o[ii, sl] = g[ii, sl] + (ev * wq[q] + bq[q])

        def body(i, carry):
            off = i * L
            ev16 = expr_v[row, pl.ds(eoff + off, L)]
            for u in range(L):
                do_row(off + u, ev16, u)
            return carry

        lax.fori_loop(0, n // L, body, 0)

        tail = n - (n // L) * L
        if tail:
            ev16 = expr_v[row, pl.ds(eoff + n - L, L)]
            for u in range(L - tail, L):
                do_row(n - L + u, ev16, u)

    def iteration(j, slot, par, do_wait_store, do_gather):
        s2 = (slot + 2) % NBUF
        if do_wait_store:
            wait_store(s2, (par + 1) % 2)
        if do_gather:
            start_gather(j + 2, s2, par)
        wait_gather(slot, par)
        compute(j, slot, par)
        start_store(j, slot, par)

    start_gather(0, 0, 0)
    start_gather(1, 1, 1)

    iteration(0, 0, 0, False, True)
    iteration(1, 1, 1, True, True)

    def hexa(qi, carry):
        jbase = 2 + 6 * qi
        for bpos in range(6):
            j = jbase + bpos
            iteration(j, (2 + bpos) % NBUF, bpos % 2, True, True)
        return carry

    lax.fori_loop(0, (NCH - 4) // 6, hexa, 0)

    iteration(NCH - 2, (NCH - 2) % NBUF, 0, True, False)
    iteration(NCH - 1, (NCH - 1) % NBUF, 1, True, False)

    wait_store((NCH - 1) % NBUF, 1)


def kernel(gene_ids, expr_values, emb_table, proj_w, proj_b):
    idx = gene_ids.astype(jnp.int32)
    expr = expr_values.astype(jnp.float32)
    w = proj_w.reshape(D).astype(jnp.float32)
    b = proj_b.reshape(D).astype(jnp.float32)
    padded = _sc_tokenize(idx, expr, emb_table.astype(jnp.float32), w, b)
    return lax.slice(padded, (0, 0, 0), (B, S, D))


# speedup vs baseline: 2.1136x; 2.1136x over previous
import functools

import jax
import jax.numpy as jnp
from jax import lax
from jax.experimental import pallas as pl
from jax.experimental.pallas import tpu as pltpu
from jax.experimental.pallas import tpu_sc as plsc

NC = 2
NS = 16
NW = NC * NS
L = 16

D = 64
B = 4096
S = 200
RPW = B // NW
NBUF = 4
SPLIT = 104

_mesh = plsc.VectorSubcoreMesh(core_axis_name="c", subcore_axis_name="s")


@functools.partial(
    pl.kernel,
    mesh=_mesh,
    compiler_params=pltpu.CompilerParams(use_tc_tiling_on_sc=False),
    out_type=jax.ShapeDtypeStruct((B, S, 2 * D), jnp.float32),
    scratch_types=[
        pltpu.VMEM((RPW, S), jnp.int32),
        pltpu.VMEM((RPW, S), jnp.float32),
        pltpu.VMEM((D,), jnp.float32),
        pltpu.VMEM((D,), jnp.float32),
        pltpu.VMEM((S, D), jnp.float32),
        pltpu.VMEM((S, D), jnp.float32),
        pltpu.VMEM((S, D), jnp.float32),
        pltpu.VMEM((S, D), jnp.float32),
        pltpu.SemaphoreType.DMA,
        pltpu.SemaphoreType.DMA,
        pltpu.SemaphoreType.DMA,
        pltpu.SemaphoreType.DMA,
        pltpu.SemaphoreType.DMA,
        pltpu.SemaphoreType.DMA,
        pltpu.SemaphoreType.DMA,
        pltpu.SemaphoreType.DMA,
    ],
)
def _sc_tokenize(idx_hbm, expr_hbm, table_hbm, w_hbm, b_hbm, out_hbm,
                 idx_v, expr_v, w_v, b_v,
                 r0, r1, r2, r3,
                 g0, g1, g2, g3, s0, s1, s2, s3):
    rows = [r0, r1, r2, r3]
    gsem = [g0, g1, g2, g3]
    ssem = [s0, s1, s2, s3]

    wid = lax.axis_index("s") * NC + lax.axis_index("c")
    row_base = wid * RPW

    pltpu.sync_copy(idx_hbm.at[pl.ds(row_base, RPW)], idx_v)
    pltpu.sync_copy(expr_hbm.at[pl.ds(row_base, RPW)], expr_v)
    pltpu.sync_copy(w_hbm, w_v)
    pltpu.sync_copy(b_hbm, b_v)

    wq = [w_v[pl.ds(q * L, L)] for q in range(4)]
    bq = [b_v[pl.ds(q * L, L)] for q in range(4)]

    def start_gather(c, s):
        pltpu.make_async_copy(
            table_hbm.at[idx_v.at[c, pl.ds(0, SPLIT)]],
            rows[s].at[pl.ds(0, SPLIT)], gsem[s]).start()
        pltpu.make_async_copy(
            table_hbm.at[idx_v.at[c, pl.ds(SPLIT, S - SPLIT)]],
            rows[s].at[pl.ds(SPLIT, S - SPLIT)], gsem[s]).start()

    def wait_gather(s):
        pltpu.make_async_copy(
            table_hbm.at[idx_v.at[0, pl.ds(0, SPLIT)]],
            rows[s].at[pl.ds(0, SPLIT)], gsem[s]).wait()
        pltpu.make_async_copy(
            table_hbm.at[idx_v.at[0, pl.ds(SPLIT, S - SPLIT)]],
            rows[s].at[pl.ds(SPLIT, S - SPLIT)], gsem[s]).wait()

    def start_store(c, s):
        dst = out_hbm.at[row_base + c, pl.ds(0, S), pl.ds(0, D)]
        pltpu.make_async_copy(rows[s], dst, ssem[s]).start()

    def wait_store(s):
        dst = out_hbm.at[row_base, pl.ds(0, S), pl.ds(0, D)]
        pltpu.make_async_copy(rows[s], dst, ssem[s]).wait()

    def compute(c, s):
        r = rows[s]

        def body(i, carry):
            off = i * L
            ev16 = expr_v[c, pl.ds(off, L)]
            for u in range(L):
                ii = off + u
                ev = jnp.full((L,), ev16[u], dtype=jnp.float32)
                for q in range(4):
                    sl = pl.ds(q * L, L)
                    r[ii, sl] = r[ii, sl] + (ev * wq[q] + bq[q])
            return carry

        lax.fori_loop(0, S // L, body, 0)

        ev16 = expr_v[c, pl.ds(S - L, L)]
        for u in range(L - (S - S // L * L), L):
            ii = S - L + u
            ev = jnp.full((L,), ev16[u], dtype=jnp.float32)
            for q in range(4):
                sl = pl.ds(q * L, L)
                r[ii, sl] = r[ii, sl] + (ev * wq[q] + bq[q])

    def iteration(j, slot, do_wait_store, do_gather):
        s2 = (slot + 2) % NBUF
        if do_wait_store:
            wait_store(s2)
        if do_gather:
            start_gather(j + 2, s2)
        wait_gather(slot)
        compute(j, slot)
        start_store(j, slot)

    start_gather(0, 0)
    start_gather(1, 1)

    iteration(0, 0, False, True)
    iteration(1, 1, False, True)

    def quad(qi, carry):
        jbase = 2 + 4 * qi
        for bpos in range(4):
            iteration(jbase + bpos, (2 + bpos) % NBUF, True, True)
        return carry

    lax.fori_loop(0, (RPW - 4) // NBUF, quad, 0)

    iteration(RPW - 2, (RPW - 2) % NBUF, True, False)
    iteration(RPW - 1, (RPW - 1) % NBUF, True, False)

    wait_store((RPW - 2) % NBUF)
    wait_store((RPW - 1) % NBUF)


def kernel(gene_ids, expr_values, emb_table, proj_w, proj_b):
    idx = gene_ids.astype(jnp.int32)
    expr = expr_values.astype(jnp.float32)
    w = proj_w.reshape(D).astype(jnp.float32)
    b = proj_b.reshape(D).astype(jnp.float32)
    padded = _sc_tokenize(idx, expr, emb_table.astype(jnp.float32), w, b)
    return lax.slice(padded, (0, 0, 0), (B, S, D))


# speedup vs baseline: 2.1340x; 1.0097x over previous
import functools

import jax
import jax.numpy as jnp
from jax import lax
from jax.experimental import pallas as pl
from jax.experimental.pallas import tpu as pltpu
from jax.experimental.pallas import tpu_sc as plsc

NC = 2
NS = 16
NW = NC * NS
L = 16

D = 64
B = 4096
S = 200
RPW = B // NW
NBUF = 4
SPLIT = 104

_mesh = plsc.VectorSubcoreMesh(core_axis_name="c", subcore_axis_name="s")


@functools.partial(
    pl.kernel,
    mesh=_mesh,
    compiler_params=pltpu.CompilerParams(use_tc_tiling_on_sc=False),
    out_type=jax.ShapeDtypeStruct((B, S, 2 * D), jnp.float32),
    scratch_types=[
        pltpu.VMEM((RPW, S), jnp.int32),
        pltpu.VMEM((RPW, S), jnp.float32),
        pltpu.VMEM((D,), jnp.float32),
        pltpu.VMEM((D,), jnp.float32),
        pltpu.VMEM((S, D), jnp.float32),
        pltpu.VMEM((S, D), jnp.float32),
        pltpu.VMEM((S, D), jnp.float32),
        pltpu.VMEM((S, D), jnp.float32),
        pltpu.SemaphoreType.DMA,
        pltpu.SemaphoreType.DMA,
        pltpu.SemaphoreType.DMA,
        pltpu.SemaphoreType.DMA,
        pltpu.SemaphoreType.DMA,
        pltpu.SemaphoreType.DMA,
        pltpu.SemaphoreType.DMA,
        pltpu.SemaphoreType.DMA,
    ],
)
def _sc_tokenize(idx_hbm, expr_hbm, table_hbm, w_hbm, b_hbm, out_hbm,
                 idx_v, expr_v, w_v, b_v,
                 r0, r1, r2, r3,
                 g0, g1, g2, g3, s0, s1, s2, s3):
    rows = [r0, r1, r2, r3]
    gsem = [g0, g1, g2, g3]
    ssem = [s0, s1, s2, s3]

    wid = lax.axis_index("s") * NC + lax.axis_index("c")
    row_base = wid * RPW

    pltpu.sync_copy(idx_hbm.at[pl.ds(row_base, RPW), pl.ds(0, S)], idx_v)
    pltpu.sync_copy(expr_hbm.at[pl.ds(row_base, RPW), pl.ds(0, S)], expr_v)
    pltpu.sync_copy(w_hbm, w_v)
    pltpu.sync_copy(b_hbm, b_v)

    wq = [w_v[pl.ds(q * L, L)] for q in range(4)]
    bq = [b_v[pl.ds(q * L, L)] for q in range(4)]

    def start_gather(c, s):
        pltpu.make_async_copy(
            table_hbm.at[idx_v.at[c, pl.ds(0, SPLIT)]],
            rows[s].at[pl.ds(0, SPLIT)], gsem[s]).start()
        pltpu.make_async_copy(
            table_hbm.at[idx_v.at[c, pl.ds(SPLIT, S - SPLIT)]],
            rows[s].at[pl.ds(SPLIT, S - SPLIT)], gsem[s]).start()

    def wait_gather(s):
        pltpu.make_async_copy(
            table_hbm.at[idx_v.at[0, pl.ds(0, SPLIT)]],
            rows[s].at[pl.ds(0, SPLIT)], gsem[s]).wait()
        pltpu.make_async_copy(
            table_hbm.at[idx_v.at[0, pl.ds(SPLIT, S - SPLIT)]],
            rows[s].at[pl.ds(SPLIT, S - SPLIT)], gsem[s]).wait()

    def start_store(c, s):
        dst = out_hbm.at[row_base + c, pl.ds(0, S), pl.ds(0, D)]
        pltpu.make_async_copy(rows[s], dst, ssem[s]).start()

    def wait_store(s):
        dst = out_hbm.at[row_base, pl.ds(0, S), pl.ds(0, D)]
        pltpu.make_async_copy(rows[s], dst, ssem[s]).wait()

    def compute(c, s):
        r = rows[s]

        def body(i, carry):
            off = i * L
            ev16 = expr_v[c, pl.ds(off, L)]
            for u in range(L):
                ii = off + u
                ev = jnp.full((L,), ev16[u], dtype=jnp.float32)
                for q in range(4):
                    sl = pl.ds(q * L, L)
                    r[ii, sl] = r[ii, sl] + (ev * wq[q] + bq[q])
            return carry

        lax.fori_loop(0, S // L, body, 0)

        ev16 = expr_v[c, pl.ds(S - L, L)]
        for u in range(L - (S - S // L * L), L):
            ii = S - L + u
            ev = jnp.full((L,), ev16[u], dtype=jnp.float32)
            for q in range(4):
                sl = pl.ds(q * L, L)
                r[ii, sl] = r[ii, sl] + (ev * wq[q] + bq[q])

    def iteration(j, slot, do_wait_store, do_gather):
        s2 = (slot + 2) % NBUF
        if do_wait_store:
            wait_store(s2)
        if do_gather:
            start_gather(j + 2, s2)
        wait_gather(slot)
        compute(j, slot)
        start_store(j, slot)

    start_gather(0, 0)
    start_gather(1, 1)

    iteration(0, 0, False, True)
    iteration(1, 1, False, True)

    def quad(qi, carry):
        jbase = 2 + 4 * qi
        for bpos in range(4):
            iteration(jbase + bpos, (2 + bpos) % NBUF, True, True)
        return carry

    lax.fori_loop(0, (RPW - 4) // NBUF, quad, 0)

    iteration(RPW - 2, (RPW - 2) % NBUF, True, False)
    iteration(RPW - 1, (RPW - 1) % NBUF, True, False)

    wait_store((RPW - 2) % NBUF)
    wait_store((RPW - 1) % NBUF)


def kernel(gene_ids, expr_values, emb_table, proj_w, proj_b):
    pad = (-S) % 256
    idx = jnp.pad(gene_ids.astype(jnp.int32), ((0, 0), (0, pad)))
    expr = jnp.pad(expr_values.astype(jnp.float32), ((0, 0), (0, pad)))
    w = proj_w.reshape(D).astype(jnp.float32)
    b = proj_b.reshape(D).astype(jnp.float32)
    padded = _sc_tokenize(idx, expr, emb_table.astype(jnp.float32), w, b)
    return lax.slice(padded, (0, 0, 0), (B, S, D))
